# Initial kernel scaffold; baseline (speedup 1.0000x reference)
import jax
import jax.numpy as jnp
from jax.experimental import pallas as pl


def kernel(x, b_pos, b_edge_index, b_edge_attr, b_type, batch, time, params):
    raise NotImplementedError("write your pallas kernel here")


# speedup vs baseline: 1.9618x; 1.9618x over previous
import functools

import numpy as np
import jax
import jax.numpy as jnp
from jax import lax
from jax.experimental import pallas as pl
from jax.experimental.pallas import tpu as pltpu
from jax.experimental.pallas import tpu_sc as plsc

HP = 80
FW = 64
CWID = 3
NGRAPH = 8
EMBW = 32

_silu = jax.nn.silu


def _dotm(a, b):
    return jnp.dot(a.astype(jnp.bfloat16), b.astype(jnp.bfloat16),
                   preferred_element_type=jnp.float32)


def _dotx(a, b):
    return jnp.dot(a, b, preferred_element_type=jnp.float32,
                   precision=lax.Precision.HIGHEST)


def _bf(v):
    return v.astype(jnp.bfloat16).astype(jnp.float32)


def _pre_body(x_ref, bpos_ref, btype_ref, time_ref, freqs_ref,
              p0_ref, p1_ref, p2_ref,
              wsx1, bsx1, wsx2, bsx2,
              wse1, bse1, wse2, bse2,
              wt1, bt1, wt2, bt2,
              wp_all, bp_all,
              h0_ref, ts_ref):
    n = x_ref.shape[0]
    for i in range(n // _NROWBLK):
        rows = pl.ds(i * _NROWBLK, _NROWBLK)
        x = x_ref[rows, :]
        xe = (_dotm(_silu(_dotm(x, wsx1[...]) + bsx1[...]), wsx2[...])
              + bsx2[...])
        bt = btype_ref[rows, :]
        be = (_dotm(_silu(_dotm(bt, wse1[...]) + bse1[...]), wse2[...])
              + bse2[...])
        h0_ref[rows, :] = (_dotx(bpos_ref[rows, :], p0_ref[...])
                           + _dotx(be, p1_ref[...])
                           + _dotx(xe, p2_ref[...]))

    e = _dotx(time_ref[...], freqs_ref[...])
    emb = jnp.concatenate([jnp.sin(e), jnp.cos(e)], axis=1)
    t1 = _silu(_dotm(emb, wt1[...]) + bt1[...])
    t2 = _dotm(t1, wt2[...]) + bt2[...]
    st = _silu(t2)
    for l in range(wp_all.shape[0]):
        ts_ref[l] = _dotm(st, wp_all[l]) + bp_all[l]


def _make_edge_body(first, last):
    def body(*refs):
        it = iter(refs)
        hs_ref = next(it)
        hd_ref = next(it)
        ea_ref = next(it)
        if first:
            wem, bem = next(it), next(it)
        w1d, w1s, w1c, w1r, b1 = (next(it) for _ in range(5))
        w2, b2, wc1, bc1, wc2, bc2 = (next(it) for _ in range(6))
        if not last:
            wu1, bu1, wu2, bu2 = (next(it) for _ in range(4))
        s_ref = next(it)
        out_ref = next(it)
        eo_ref = None if last else next(it)

        hs = hs_ref[...]
        hd = hd_ref[...]
        d = hs - hd
        lane = lax.broadcasted_iota(jnp.int32, (1, HP), 1)
        maskc = (lane < CWID).astype(jnp.float32)
        dm = d * maskc
        rd = jnp.sum(dm * dm, axis=1, keepdims=True)
        if first:
            ea = _dotm(ea_ref[...], wem[...]) + bem[...]
        else:
            ea = ea_ref[...]
        e1 = (_dotm(hd, w1d[...]) + _dotm(hs, w1s[...])
              + _dotm(ea, w1c[...])
              + _bf(rd) * _bf(w1r[...])
              + b1[...])
        m = _silu(_dotm(_silu(e1), w2[...]) + b2[...])
        cw = _dotm(_silu(_dotm(m, wc1[...]) + bc1[...]), wc2[...]) + bc2[...]
        out_ref[...] = cw * dm + _dotx(m, s_ref[...])
        if eo_ref is not None:
            eo_ref[...] = (_dotm(_silu(_dotm(m, wu1[...]) + bu1[...]),
                                 wu2[...]) + bu2[...])
    return body


_NROWBLK = 2000


def _node_body(h_ref, a0_ref, a1_ref, batch_ref, ts_ref, st_ref, s_ref,
               wn1f, wn1m, bn1, wn2, bn2, nw, nb, wf1, bf1, wf2, bf2,
               hout_ref, hid_ref):
    n = h_ref.shape[0]
    nblk = n // _NROWBLK
    gl = lax.broadcasted_iota(jnp.int32, (1, NGRAPH), 1)
    lane = lax.broadcasted_iota(jnp.int32, (1, HP), 1)
    maskc = (lane < CWID).astype(jnp.float32)
    ts = ts_ref[...]

    cnt_row = jnp.zeros((1, NGRAPH), jnp.float32)
    sum_row = jnp.zeros((1, NGRAPH), jnp.float32)
    sq_row = jnp.zeros((1, NGRAPH), jnp.float32)
    for i in range(nblk):
        rows = pl.ds(i * _NROWBLK, _NROWBLK)
        h = h_ref[rows, :]
        agg = a0_ref[rows, :] + a1_ref[rows, :]
        feats = _dotx(h, st_ref[...])
        m_i = _dotx(agg, st_ref[...])
        pre = _dotm(feats, wn1f[...]) + _dotm(m_i, wn1m[...]) + bn1[...]
        hidden = _dotm(_silu(pre), wn2[...]) + bn2[...]
        oh = (batch_ref[rows, :] == gl).astype(jnp.float32)
        scale = _dotx(oh, ts[:, :FW])
        shift = _dotx(oh, ts[:, FW:])
        hidden = hidden * (scale + 1.0) + shift
        hid_ref[rows, :] = hidden
        cnt_row = cnt_row + jnp.sum(oh, axis=0, keepdims=True)
        rowsum = jnp.sum(hidden, axis=1, keepdims=True)
        sum_row = sum_row + jnp.sum(oh * rowsum, axis=0, keepdims=True)
        rowsq = jnp.sum(hidden * hidden, axis=1, keepdims=True)
        sq_row = sq_row + jnp.sum(oh * rowsq, axis=0, keepdims=True)

    norm_row = jnp.maximum(cnt_row, 1.0) * float(FW)
    mean_row = sum_row / norm_row
    var_row = sq_row / norm_row - mean_row * mean_row
    inv_row = 1.0 / jnp.sqrt(var_row + 1e-5)

    for i in range(nblk):
        rows = pl.ds(i * _NROWBLK, _NROWBLK)
        oh = (batch_ref[rows, :] == gl).astype(jnp.float32)
        mean_n = jnp.sum(oh * mean_row, axis=1, keepdims=True)
        inv_n = jnp.sum(oh * inv_row, axis=1, keepdims=True)
        cen = hid_ref[rows, :] - mean_n
        fn = cen * inv_n * nw[...] + nb[...]
        ff = _dotm(jax.nn.gelu(_dotm(fn, wf1[...]) + bf1[...]),
                   wf2[...]) + bf2[...]
        feats_new = ff + fn
        agg = a0_ref[rows, :] + a1_ref[rows, :]
        hout_ref[rows, :] = ((h_ref[rows, :] + agg) * maskc
                             + _dotx(feats_new, s_ref[...]))


def _dec_body(h_ref, st_ref, wd1, bd1, wd2, bd2, out_ref):
    n = h_ref.shape[0]
    for i in range(n // _NROWBLK):
        rows = pl.ds(i * _NROWBLK, _NROWBLK)
        xo = _dotx(h_ref[rows, :], st_ref[...])
        out_ref[rows, :] = (_dotm(_silu(_dotm(xo, wd1[...]) + bd1[...]),
                                  wd2[...]) + bd2[...])


_GCH = 80


def _sc_gather(table, idx):
    m = idx.shape[0]
    info = plsc.get_sparse_core_info()
    nw = info.num_cores * info.num_subcores
    per_w = m // nw
    assert per_w * nw == m and per_w % _GCH == 0
    nch = per_w // _GCH
    mesh = plsc.VectorSubcoreMesh(core_axis_name="c", subcore_axis_name="s")

    @functools.partial(
        pl.kernel, mesh=mesh,
        out_type=jax.ShapeDtypeStruct((m, HP), jnp.float32),
        compiler_params=pltpu.CompilerParams(use_tc_tiling_on_sc=False),
        scratch_types=[
            pltpu.VMEM((_GCH,), jnp.int32),
            pltpu.VMEM((_GCH, HP), jnp.float32),
            pltpu.SemaphoreType.DMA,
        ],
    )
    def k(table_hbm, idx_hbm, out_hbm, idx_v, rows_v, sem):
        wid = lax.axis_index("s") * info.num_cores + lax.axis_index("c")
        base = wid * per_w

        def body(i, carry):
            off = base + i * _GCH
            pltpu.sync_copy(idx_hbm.at[pl.ds(off, _GCH)], idx_v)
            pltpu.async_copy(table_hbm.at[idx_v], rows_v, sem).wait()
            pltpu.sync_copy(rows_v, out_hbm.at[pl.ds(off, _GCH)])
            return carry

        lax.fori_loop(0, nch, body, 0)

    return k(table, idx)


def _sc_scatter(vals, dst, zeros, n_nodes):
    e = vals.shape[0]
    info = plsc.get_sparse_core_info()
    nc, ns = info.num_cores, info.num_subcores
    nw = nc * ns
    per_w = e // nw
    assert per_w * nw == e and per_w % _GCH == 0
    nch = per_w // _GCH
    rows_pt = n_nodes // ns
    assert rows_pt * ns == n_nodes
    mesh = plsc.VectorSubcoreMesh(core_axis_name="c", subcore_axis_name="s")

    @functools.partial(
        pl.kernel, mesh=mesh,
        out_type=jax.ShapeDtypeStruct((nc, n_nodes, HP), jnp.float32),
        compiler_params=pltpu.CompilerParams(use_tc_tiling_on_sc=False),
        scratch_types=[
            pltpu.VMEM((_GCH, HP), jnp.float32),
            pltpu.VMEM((_GCH,), jnp.int32),
            pltpu.VMEM((rows_pt, HP), jnp.float32),
            pltpu.VMEM_SHARED((n_nodes, HP), jnp.float32),
        ],
    )
    def k(vals_hbm, dst_hbm, zeros_hbm, out_hbm, vals_v, idx_v, buf_v, acc):
        c = lax.axis_index("c")
        s = lax.axis_index("s")
        wid = s * nc + c
        base = wid * per_w
        pltpu.sync_copy(zeros_hbm, buf_v)
        pltpu.sync_copy(buf_v, acc.at[pl.ds(s * rows_pt, rows_pt)])
        plsc.subcore_barrier()

        def body(i, carry):
            off = base + i * _GCH
            pltpu.sync_copy(vals_hbm.at[pl.ds(off, _GCH)], vals_v)
            pltpu.sync_copy(dst_hbm.at[pl.ds(off, _GCH)], idx_v)
            pltpu.sync_copy(vals_v, acc.at[idx_v], add=True)
            return carry

        lax.fori_loop(0, nch, body, 0)
        plsc.subcore_barrier()
        pltpu.sync_copy(acc.at[pl.ds(s * rows_pt, rows_pt)], buf_v)
        pltpu.sync_copy(buf_v, out_hbm.at[c, pl.ds(s * rows_pt, rows_pt)])

    return k(vals, dst, zeros)


_EBLK = 3200


def _full_spec(a):
    return pl.BlockSpec(a.shape, lambda i: (0,) * a.ndim)


def kernel(x, b_pos, b_edge_index, b_edge_attr, b_type, batch, time, params):
    n = x.shape[0]
    e = b_edge_index.shape[1]
    n_layers = len(params["layers"])
    src = b_edge_index[0]
    dst = b_edge_index[1]
    idx2 = jnp.concatenate([src, dst])

    f32 = jnp.float32
    eye = np.eye(HP, dtype=np.float32)
    p0 = jnp.asarray(eye[:CWID])
    p1 = jnp.asarray(eye[CWID:CWID + EMBW])
    p2 = jnp.asarray(eye[CWID + EMBW:CWID + 2 * EMBW])
    smat = jnp.asarray(eye[CWID:CWID + FW])
    stmat = smat.T

    half = FW // 2
    freqs = np.exp(np.arange(half, dtype=np.float32)
                   * np.float32(-np.log(10000.0) / (half - 1)),
                   dtype=np.float32)[None, :]

    def row(v):
        return v.reshape(1, -1)

    pm = params
    wp_all = jnp.stack([lp["time_proj"]["w"] for lp in pm["layers"]])
    bp_all = jnp.stack([row(lp["time_proj"]["b"]) for lp in pm["layers"]])

    h0, ts_all = pl.pallas_call(
        _pre_body,
        out_shape=(jax.ShapeDtypeStruct((n, HP), f32),
                   jax.ShapeDtypeStruct((n_layers, NGRAPH, 2 * FW), f32)),
    )(x, b_pos, b_type, time.reshape(-1, 1), jnp.asarray(freqs),
      p0, p1, p2,
      pm["ss_mlp"][0]["w"], row(pm["ss_mlp"][0]["b"]),
      pm["ss_mlp"][1]["w"], row(pm["ss_mlp"][1]["b"]),
      pm["ss_emb"][0]["w"], row(pm["ss_emb"][0]["b"]),
      pm["ss_emb"][1]["w"], row(pm["ss_emb"][1]["b"]),
      pm["time_mlp"][0]["w"], row(pm["time_mlp"][0]["b"]),
      pm["time_mlp"][1]["w"], row(pm["time_mlp"][1]["b"]),
      wp_all, bp_all)

    zeros = jnp.zeros((n // 16, HP), f32)
    batch_col = batch.reshape(-1, 1)
    h = h0
    ea = b_edge_attr
    nblk = e // _EBLK

    for li, lp in enumerate(params["layers"]):
        first = li == 0
        last = li == n_layers - 1
        w1 = lp["edge_mlp"][0]["w"]
        b1 = row(lp["edge_mlp"][0]["b"])
        kdim = w1.shape[1]
        w1d = jnp.zeros((HP, kdim), f32).at[CWID:CWID + FW].set(w1[:FW])
        w1s = jnp.zeros((HP, kdim), f32).at[CWID:CWID + FW].set(w1[FW:2 * FW])
        w1c = w1[2 * FW:2 * FW + EMBW]
        w1r = row(w1[2 * FW + EMBW])
        edge_args = []
        if first:
            edge_args += [pm["edge_emb"]["w"], row(pm["edge_emb"]["b"])]
        edge_args += [
            w1d, w1s, w1c, w1r, b1,
            lp["edge_mlp"][1]["w"], row(lp["edge_mlp"][1]["b"]),
            lp["coors_mlp"][0]["w"], row(lp["coors_mlp"][0]["b"]),
            lp["coors_mlp"][1]["w"], row(lp["coors_mlp"][1]["b"]),
        ]
        if not last:
            edge_args += [
                lp["edge_upd"][0]["w"], row(lp["edge_upd"][0]["b"]),
                lp["edge_upd"][1]["w"], row(lp["edge_upd"][1]["b"]),
            ]
        edge_args.append(smat)

        g = _sc_gather(h, idx2)
        hs = g[:e]
        hd = g[e:]

        eaw = ea.shape[1]
        in_specs = ([pl.BlockSpec((_EBLK, HP), lambda i: (i, 0)),
                     pl.BlockSpec((_EBLK, HP), lambda i: (i, 0)),
                     pl.BlockSpec((_EBLK, eaw), lambda i: (i, 0))]
                    + [_full_spec(a) for a in edge_args])
        if last:
            out_shape = jax.ShapeDtypeStruct((e, HP), f32)
            out_specs = pl.BlockSpec((_EBLK, HP), lambda i: (i, 0))
        else:
            out_shape = (jax.ShapeDtypeStruct((e, HP), f32),
                         jax.ShapeDtypeStruct((e, EMBW), f32))
            out_specs = (pl.BlockSpec((_EBLK, HP), lambda i: (i, 0)),
                         pl.BlockSpec((_EBLK, EMBW), lambda i: (i, 0)))
        res = pl.pallas_call(
            _make_edge_body(first, last),
            grid=(nblk,),
            in_specs=in_specs,
            out_specs=out_specs,
            out_shape=out_shape,
            compiler_params=pltpu.CompilerParams(
                dimension_semantics=("arbitrary",)),
        )(hs, hd, ea, *edge_args)
        if last:
            out_edge = res
        else:
            out_edge, ea = res

        parts = _sc_scatter(out_edge, dst, zeros, n)

        node_args = [
            lp["node_mlp"][0]["w"][:FW], lp["node_mlp"][0]["w"][FW:],
            row(lp["node_mlp"][0]["b"]),
            lp["node_mlp"][1]["w"], row(lp["node_mlp"][1]["b"]),
            row(lp["ff_norm"]["w"]), row(lp["ff_norm"]["b"]),
            lp["ff"][0]["w"], row(lp["ff"][0]["b"]),
            lp["ff"][1]["w"], row(lp["ff"][1]["b"]),
        ]
        h = pl.pallas_call(
            _node_body,
            out_shape=jax.ShapeDtypeStruct((n, HP), f32),
            scratch_shapes=[pltpu.VMEM((n, FW), f32)],
        )(h, parts[0], parts[1], batch_col, ts_all[li], stmat, smat, *node_args)

    dec = pl.pallas_call(
        _dec_body,
        out_shape=jax.ShapeDtypeStruct((n, x.shape[1] + 3), f32),
    )(h, stmat,
      pm["ss_decode"][0]["w"], row(pm["ss_decode"][0]["b"]),
      pm["ss_decode"][1]["w"], row(pm["ss_decode"][1]["b"]))
    return dec[:, :x.shape[1]], dec[:, x.shape[1]:]


# speedup vs baseline: 2.1726x; 1.1074x over previous
import functools

import numpy as np
import jax
import jax.numpy as jnp
from jax import lax
from jax.experimental import pallas as pl
from jax.experimental.pallas import tpu as pltpu
from jax.experimental.pallas import tpu_sc as plsc

HP = 80
FW = 64
CWID = 3
NGRAPH = 8
EMBW = 32

_silu = jax.nn.silu


def _dotm(a, b):
    return jnp.dot(a.astype(jnp.bfloat16), b.astype(jnp.bfloat16),
                   preferred_element_type=jnp.float32)


def _dotx(a, b):
    return jnp.dot(a, b, preferred_element_type=jnp.float32,
                   precision=lax.Precision.HIGHEST)


def _bf(v):
    return v.astype(jnp.bfloat16).astype(jnp.float32)


def _pre_body(x_ref, bpos_ref, btype_ref, time_ref, freqs_ref,
              p0_ref, p1_ref, p2_ref,
              wsx1, bsx1, wsx2, bsx2,
              wse1, bse1, wse2, bse2,
              wt1, bt1, wt2, bt2,
              wp_all, bp_all,
              h0_ref, ts_ref):
    n = x_ref.shape[0]
    for i in range(n // _NROWBLK):
        rows = pl.ds(i * _NROWBLK, _NROWBLK)
        x = x_ref[rows, :]
        xe = (_dotm(_silu(_dotm(x, wsx1[...]) + bsx1[...]), wsx2[...])
              + bsx2[...])
        bt = btype_ref[rows, :]
        be = (_dotm(_silu(_dotm(bt, wse1[...]) + bse1[...]), wse2[...])
              + bse2[...])
        h0_ref[rows, :] = (_dotx(bpos_ref[rows, :], p0_ref[...])
                           + _dotx(be, p1_ref[...])
                           + _dotx(xe, p2_ref[...]))

    e = _dotx(time_ref[...], freqs_ref[...])
    emb = jnp.concatenate([jnp.sin(e), jnp.cos(e)], axis=1)
    t1 = _silu(_dotm(emb, wt1[...]) + bt1[...])
    t2 = _dotm(t1, wt2[...]) + bt2[...]
    st = _silu(t2)
    for l in range(wp_all.shape[0]):
        ts_ref[l] = _dotm(st, wp_all[l]) + bp_all[l]


def _make_edge_body(first, last):
    def body(*refs):
        it = iter(refs)
        hs_ref = next(it)
        hd_ref = next(it)
        ea_ref = next(it)
        if first:
            wem, bem = next(it), next(it)
        w1d, w1s, w1c, w1r, b1 = (next(it) for _ in range(5))
        w2, b2, wc1, bc1, wc2, bc2 = (next(it) for _ in range(6))
        if not last:
            wu1, bu1, wu2, bu2 = (next(it) for _ in range(4))
        s_ref = next(it)
        out_ref = next(it)
        eo_ref = None if last else next(it)

        hs = hs_ref[...]
        hd = hd_ref[...]
        d = hs - hd
        lane = lax.broadcasted_iota(jnp.int32, (1, HP), 1)
        maskc = (lane < CWID).astype(jnp.float32)
        dm = d * maskc
        rd = jnp.sum(dm * dm, axis=1, keepdims=True)
        if first:
            ea = _dotm(ea_ref[...], wem[...]) + bem[...]
        else:
            ea = ea_ref[...]
        e1 = (_dotm(hd, w1d[...]) + _dotm(hs, w1s[...])
              + _dotm(ea, w1c[...])
              + _bf(rd) * _bf(w1r[...])
              + b1[...])
        m = _silu(_dotm(_silu(e1), w2[...]) + b2[...])
        cw = _dotm(_silu(_dotm(m, wc1[...]) + bc1[...]), wc2[...]) + bc2[...]
        out_ref[...] = cw * dm + _dotx(m, s_ref[...])
        if eo_ref is not None:
            eo_ref[...] = (_dotm(_silu(_dotm(m, wu1[...]) + bu1[...]),
                                 wu2[...]) + bu2[...])
    return body


_NROWBLK = 2000


def _node_body(h_ref, a0_ref, a1_ref, batch_ref, ts_ref, st_ref, s_ref,
               wn1f, wn1m, bn1, wn2, bn2, nw, nb, wf1, bf1, wf2, bf2,
               hout_ref, hid_ref):
    n = h_ref.shape[0]
    nblk = n // _NROWBLK
    gl = lax.broadcasted_iota(jnp.int32, (1, NGRAPH), 1)
    lane = lax.broadcasted_iota(jnp.int32, (1, HP), 1)
    maskc = (lane < CWID).astype(jnp.float32)
    ts = ts_ref[...]

    cnt_row = jnp.zeros((1, NGRAPH), jnp.float32)
    sum_row = jnp.zeros((1, NGRAPH), jnp.float32)
    sq_row = jnp.zeros((1, NGRAPH), jnp.float32)
    for i in range(nblk):
        rows = pl.ds(i * _NROWBLK, _NROWBLK)
        h = h_ref[rows, :]
        agg = a0_ref[rows, :] + a1_ref[rows, :]
        feats = _dotx(h, st_ref[...])
        m_i = _dotx(agg, st_ref[...])
        pre = _dotm(feats, wn1f[...]) + _dotm(m_i, wn1m[...]) + bn1[...]
        hidden = _dotm(_silu(pre), wn2[...]) + bn2[...]
        oh = (batch_ref[rows, :] == gl).astype(jnp.float32)
        scale = _dotx(oh, ts[:, :FW])
        shift = _dotx(oh, ts[:, FW:])
        hidden = hidden * (scale + 1.0) + shift
        hid_ref[rows, :] = hidden
        cnt_row = cnt_row + jnp.sum(oh, axis=0, keepdims=True)
        rowsum = jnp.sum(hidden, axis=1, keepdims=True)
        sum_row = sum_row + jnp.sum(oh * rowsum, axis=0, keepdims=True)
        rowsq = jnp.sum(hidden * hidden, axis=1, keepdims=True)
        sq_row = sq_row + jnp.sum(oh * rowsq, axis=0, keepdims=True)

    norm_row = jnp.maximum(cnt_row, 1.0) * float(FW)
    mean_row = sum_row / norm_row
    var_row = sq_row / norm_row - mean_row * mean_row
    inv_row = 1.0 / jnp.sqrt(var_row + 1e-5)

    for i in range(nblk):
        rows = pl.ds(i * _NROWBLK, _NROWBLK)
        oh = (batch_ref[rows, :] == gl).astype(jnp.float32)
        mean_n = jnp.sum(oh * mean_row, axis=1, keepdims=True)
        inv_n = jnp.sum(oh * inv_row, axis=1, keepdims=True)
        cen = hid_ref[rows, :] - mean_n
        fn = cen * inv_n * nw[...] + nb[...]
        ff = _dotm(jax.nn.gelu(_dotm(fn, wf1[...]) + bf1[...]),
                   wf2[...]) + bf2[...]
        feats_new = ff + fn
        agg = a0_ref[rows, :] + a1_ref[rows, :]
        hout_ref[rows, :] = ((h_ref[rows, :] + agg) * maskc
                             + _dotx(feats_new, s_ref[...]))


def _dec_body(h_ref, st_ref, wd1, bd1, wd2, bd2, out_ref):
    n = h_ref.shape[0]
    for i in range(n // _NROWBLK):
        rows = pl.ds(i * _NROWBLK, _NROWBLK)
        xo = _dotx(h_ref[rows, :], st_ref[...])
        out_ref[rows, :] = (_dotm(_silu(_dotm(xo, wd1[...]) + bd1[...]),
                                  wd2[...]) + bd2[...])


_GCH = 80


def _sc_gather(table, idx):
    m = idx.shape[0]
    info = plsc.get_sparse_core_info()
    nw = info.num_cores * info.num_subcores
    per_w = m // nw
    assert per_w * nw == m and per_w % _GCH == 0
    nch = per_w // _GCH
    mesh = plsc.VectorSubcoreMesh(core_axis_name="c", subcore_axis_name="s")

    assert nch % 2 == 0

    @functools.partial(
        pl.kernel, mesh=mesh,
        out_type=jax.ShapeDtypeStruct((m, HP), jnp.float32),
        compiler_params=pltpu.CompilerParams(use_tc_tiling_on_sc=False),
        scratch_types=[
            pltpu.VMEM((per_w,), jnp.int32),
            pltpu.VMEM((_GCH, HP), jnp.float32),
            pltpu.VMEM((_GCH, HP), jnp.float32),
            pltpu.SemaphoreType.DMA,
            pltpu.SemaphoreType.DMA,
        ],
    )
    def k(table_hbm, idx_hbm, out_hbm, idx_v, rows_a, rows_b, sem_a, sem_b):
        wid = lax.axis_index("s") * info.num_cores + lax.axis_index("c")
        base = wid * per_w
        pltpu.sync_copy(idx_hbm.at[pl.ds(base, per_w)], idx_v)

        def body(j, carry):
            i0 = 2 * j * _GCH
            i1 = i0 + _GCH
            c0 = pltpu.async_copy(
                table_hbm.at[idx_v.at[pl.ds(i0, _GCH)]], rows_a, sem_a)
            c1 = pltpu.async_copy(
                table_hbm.at[idx_v.at[pl.ds(i1, _GCH)]], rows_b, sem_b)
            c0.wait()
            pltpu.sync_copy(rows_a, out_hbm.at[pl.ds(base + i0, _GCH)])
            c1.wait()
            pltpu.sync_copy(rows_b, out_hbm.at[pl.ds(base + i1, _GCH)])
            return carry

        lax.fori_loop(0, nch // 2, body, 0)

    return k(table, idx)


def _sc_scatter(vals, dst, zeros, n_nodes):
    e = vals.shape[0]
    info = plsc.get_sparse_core_info()
    nc, ns = info.num_cores, info.num_subcores
    nw = nc * ns
    per_w = e // nw
    assert per_w * nw == e and per_w % _GCH == 0
    nch = per_w // _GCH
    rows_pt = n_nodes // ns
    assert rows_pt * ns == n_nodes
    mesh = plsc.VectorSubcoreMesh(core_axis_name="c", subcore_axis_name="s")

    @functools.partial(
        pl.kernel, mesh=mesh,
        out_type=jax.ShapeDtypeStruct((nc, n_nodes, HP), jnp.float32),
        compiler_params=pltpu.CompilerParams(use_tc_tiling_on_sc=False),
        scratch_types=[
            pltpu.VMEM((_GCH, HP), jnp.float32),
            pltpu.VMEM((_GCH,), jnp.int32),
            pltpu.VMEM((rows_pt, HP), jnp.float32),
            pltpu.VMEM_SHARED((n_nodes, HP), jnp.float32),
        ],
    )
    def k(vals_hbm, dst_hbm, zeros_hbm, out_hbm, vals_v, idx_v, buf_v, acc):
        c = lax.axis_index("c")
        s = lax.axis_index("s")
        wid = s * nc + c
        base = wid * per_w
        pltpu.sync_copy(zeros_hbm, buf_v)
        pltpu.sync_copy(buf_v, acc.at[pl.ds(s * rows_pt, rows_pt)])
        plsc.subcore_barrier()

        def body(i, carry):
            off = base + i * _GCH
            pltpu.sync_copy(vals_hbm.at[pl.ds(off, _GCH)], vals_v)
            pltpu.sync_copy(dst_hbm.at[pl.ds(off, _GCH)], idx_v)
            pltpu.sync_copy(vals_v, acc.at[idx_v], add=True)
            return carry

        lax.fori_loop(0, nch, body, 0)
        plsc.subcore_barrier()
        pltpu.sync_copy(acc.at[pl.ds(s * rows_pt, rows_pt)], buf_v)
        pltpu.sync_copy(buf_v, out_hbm.at[c, pl.ds(s * rows_pt, rows_pt)])

    return k(vals, dst, zeros)


_EBLK = 3200


def _full_spec(a):
    return pl.BlockSpec(a.shape, lambda i: (0,) * a.ndim)


def kernel(x, b_pos, b_edge_index, b_edge_attr, b_type, batch, time, params):
    n = x.shape[0]
    e = b_edge_index.shape[1]
    n_layers = len(params["layers"])
    src = b_edge_index[0]
    dst = b_edge_index[1]
    idx2 = jnp.concatenate([src, dst])

    f32 = jnp.float32
    eye = np.eye(HP, dtype=np.float32)
    p0 = jnp.asarray(eye[:CWID])
    p1 = jnp.asarray(eye[CWID:CWID + EMBW])
    p2 = jnp.asarray(eye[CWID + EMBW:CWID + 2 * EMBW])
    smat = jnp.asarray(eye[CWID:CWID + FW])
    stmat = smat.T

    half = FW // 2
    freqs = np.exp(np.arange(half, dtype=np.float32)
                   * np.float32(-np.log(10000.0) / (half - 1)),
                   dtype=np.float32)[None, :]

    def row(v):
        return v.reshape(1, -1)

    pm = params
    wp_all = jnp.stack([lp["time_proj"]["w"] for lp in pm["layers"]])
    bp_all = jnp.stack([row(lp["time_proj"]["b"]) for lp in pm["layers"]])

    h0, ts_all = pl.pallas_call(
        _pre_body,
        out_shape=(jax.ShapeDtypeStruct((n, HP), f32),
                   jax.ShapeDtypeStruct((n_layers, NGRAPH, 2 * FW), f32)),
    )(x, b_pos, b_type, time.reshape(-1, 1), jnp.asarray(freqs),
      p0, p1, p2,
      pm["ss_mlp"][0]["w"], row(pm["ss_mlp"][0]["b"]),
      pm["ss_mlp"][1]["w"], row(pm["ss_mlp"][1]["b"]),
      pm["ss_emb"][0]["w"], row(pm["ss_emb"][0]["b"]),
      pm["ss_emb"][1]["w"], row(pm["ss_emb"][1]["b"]),
      pm["time_mlp"][0]["w"], row(pm["time_mlp"][0]["b"]),
      pm["time_mlp"][1]["w"], row(pm["time_mlp"][1]["b"]),
      wp_all, bp_all)

    zeros = jnp.zeros((n // 16, HP), f32)
    batch_col = batch.reshape(-1, 1)
    h = h0
    ea = b_edge_attr
    nblk = e // _EBLK

    for li, lp in enumerate(params["layers"]):
        first = li == 0
        last = li == n_layers - 1
        w1 = lp["edge_mlp"][0]["w"]
        b1 = row(lp["edge_mlp"][0]["b"])
        kdim = w1.shape[1]
        w1d = jnp.zeros((HP, kdim), f32).at[CWID:CWID + FW].set(w1[:FW])
        w1s = jnp.zeros((HP, kdim), f32).at[CWID:CWID + FW].set(w1[FW:2 * FW])
        w1c = w1[2 * FW:2 * FW + EMBW]
        w1r = row(w1[2 * FW + EMBW])
        edge_args = []
        if first:
            edge_args += [pm["edge_emb"]["w"], row(pm["edge_emb"]["b"])]
        edge_args += [
            w1d, w1s, w1c, w1r, b1,
            lp["edge_mlp"][1]["w"], row(lp["edge_mlp"][1]["b"]),
            lp["coors_mlp"][0]["w"], row(lp["coors_mlp"][0]["b"]),
            lp["coors_mlp"][1]["w"], row(lp["coors_mlp"][1]["b"]),
        ]
        if not last:
            edge_args += [
                lp["edge_upd"][0]["w"], row(lp["edge_upd"][0]["b"]),
                lp["edge_upd"][1]["w"], row(lp["edge_upd"][1]["b"]),
            ]
        edge_args.append(smat)

        g = _sc_gather(h, idx2)
        hs = g[:e]
        hd = g[e:]

        eaw = ea.shape[1]
        in_specs = ([pl.BlockSpec((_EBLK, HP), lambda i: (i, 0)),
                     pl.BlockSpec((_EBLK, HP), lambda i: (i, 0)),
                     pl.BlockSpec((_EBLK, eaw), lambda i: (i, 0))]
                    + [_full_spec(a) for a in edge_args])
        if last:
            out_shape = jax.ShapeDtypeStruct((e, HP), f32)
            out_specs = pl.BlockSpec((_EBLK, HP), lambda i: (i, 0))
        else:
            out_shape = (jax.ShapeDtypeStruct((e, HP), f32),
                         jax.ShapeDtypeStruct((e, EMBW), f32))
            out_specs = (pl.BlockSpec((_EBLK, HP), lambda i: (i, 0)),
                         pl.BlockSpec((_EBLK, EMBW), lambda i: (i, 0)))
        res = pl.pallas_call(
            _make_edge_body(first, last),
            grid=(nblk,),
            in_specs=in_specs,
            out_specs=out_specs,
            out_shape=out_shape,
            compiler_params=pltpu.CompilerParams(
                dimension_semantics=("arbitrary",)),
        )(hs, hd, ea, *edge_args)
        if last:
            out_edge = res
        else:
            out_edge, ea = res

        parts = _sc_scatter(out_edge, dst, zeros, n)

        node_args = [
            lp["node_mlp"][0]["w"][:FW], lp["node_mlp"][0]["w"][FW:],
            row(lp["node_mlp"][0]["b"]),
            lp["node_mlp"][1]["w"], row(lp["node_mlp"][1]["b"]),
            row(lp["ff_norm"]["w"]), row(lp["ff_norm"]["b"]),
            lp["ff"][0]["w"], row(lp["ff"][0]["b"]),
            lp["ff"][1]["w"], row(lp["ff"][1]["b"]),
        ]
        h = pl.pallas_call(
            _node_body,
            out_shape=jax.ShapeDtypeStruct((n, HP), f32),
            scratch_shapes=[pltpu.VMEM((n, FW), f32)],
        )(h, parts[0], parts[1], batch_col, ts_all[li], stmat, smat, *node_args)

    dec = pl.pallas_call(
        _dec_body,
        out_shape=jax.ShapeDtypeStruct((n, x.shape[1] + 3), f32),
    )(h, stmat,
      pm["ss_decode"][0]["w"], row(pm["ss_decode"][0]["b"]),
      pm["ss_decode"][1]["w"], row(pm["ss_decode"][1]["b"]))
    return dec[:, :x.shape[1]], dec[:, x.shape[1]:]
